# baseline (speedup 1.0000x reference)
import functools

import jax
import jax.numpy as jnp
from jax.experimental import pallas as pl
from jax.experimental.pallas import tpu as pltpu

_ROUTER_DIM = 256
_NUM_EXPERTS = 64
_TOP_K = 8
_BLK = 512


def _router_kernel(x_ref, wt_ref, b_in_ref, b_mat_ref, wg_ref, b_gate_ref,
                   idx_ref, w_ref):
    xb = x_ref[...]
    xp = jnp.dot(xb, wt_ref[...], preferred_element_type=jnp.float32)
    xp = xp + b_in_ref[...]
    h = jnp.tanh(jnp.dot(xp, b_mat_ref[...],
                         preferred_element_type=jnp.float32)) * 0.1
    logits = jnp.dot(h, wg_ref[...], preferred_element_type=jnp.float32)
    logits = logits + b_gate_ref[...]

    blk = logits.shape[0]
    iota = jax.lax.broadcasted_iota(jnp.int32, (blk, _NUM_EXPERTS), 1)
    vals = logits
    top_vals = []
    top_idx = []
    for _ in range(_TOP_K):
        m = jnp.max(vals, axis=1, keepdims=True)
        idx = jnp.min(jnp.where(vals == m, iota, _NUM_EXPERTS),
                      axis=1, keepdims=True)
        top_vals.append(m)
        top_idx.append(idx)
        vals = jnp.where(iota == idx, -jnp.inf, vals)
    tv = jnp.concatenate(top_vals, axis=1)
    ti = jnp.concatenate(top_idx, axis=1)
    e = jnp.exp(tv - tv[:, 0:1])
    w = e / jnp.sum(e, axis=1, keepdims=True)
    idx_ref[...] = ti
    w_ref[...] = w


@jax.jit
def kernel(x, W_in, b_in, tau, A, B, W_gate, b_gate):
    del tau, A
    n_tokens, hidden = x.shape
    wt = W_in.T
    wg = W_gate.T
    b_in2 = b_in.reshape(1, _ROUTER_DIM)
    b_gate2 = b_gate.reshape(1, _NUM_EXPERTS)
    grid = (n_tokens // _BLK,)
    out_idx, out_w = pl.pallas_call(
        _router_kernel,
        grid=grid,
        in_specs=[
            pl.BlockSpec((_BLK, hidden), lambda i: (i, 0)),
            pl.BlockSpec((hidden, _ROUTER_DIM), lambda i: (0, 0)),
            pl.BlockSpec((1, _ROUTER_DIM), lambda i: (0, 0)),
            pl.BlockSpec((_ROUTER_DIM, _ROUTER_DIM), lambda i: (0, 0)),
            pl.BlockSpec((_ROUTER_DIM, _NUM_EXPERTS), lambda i: (0, 0)),
            pl.BlockSpec((1, _NUM_EXPERTS), lambda i: (0, 0)),
        ],
        out_specs=[
            pl.BlockSpec((_BLK, _TOP_K), lambda i: (i, 0)),
            pl.BlockSpec((_BLK, _TOP_K), lambda i: (i, 0)),
        ],
        out_shape=[
            jax.ShapeDtypeStruct((n_tokens, _TOP_K), jnp.int32),
            jax.ShapeDtypeStruct((n_tokens, _TOP_K), jnp.float32),
        ],
        compiler_params=pltpu.CompilerParams(
            dimension_semantics=("parallel",),
        ),
    )(x, wt, b_in2, B, wg, b_gate2)
    return out_idx, out_w


# speedup vs baseline: 1.1275x; 1.1275x over previous
import functools

import jax
import jax.numpy as jnp
from jax.experimental import pallas as pl
from jax.experimental.pallas import tpu as pltpu

_ROUTER_DIM = 256
_NUM_EXPERTS = 64
_TOP_K = 8
_BLK = 1024


def _router_kernel(x_ref, wt_ref, b_in_ref, b_mat_ref, wg_ref, b_gate_ref,
                   idx_ref, w_ref):
    xb = x_ref[...]
    xp = jnp.dot(xb, wt_ref[...], preferred_element_type=jnp.float32)
    xp = xp + b_in_ref[...]
    h = jnp.tanh(jnp.dot(xp, b_mat_ref[...],
                         preferred_element_type=jnp.float32)) * 0.1
    logits = jnp.dot(h, wg_ref[...], preferred_element_type=jnp.float32)
    logits = logits + b_gate_ref[...]

    blk = logits.shape[0]
    iota = jax.lax.broadcasted_iota(jnp.int32, (blk, _NUM_EXPERTS), 1)
    vals = logits
    top_vals = []
    top_idx = []
    for _ in range(_TOP_K):
        m = jnp.max(vals, axis=1, keepdims=True)
        idx = jnp.min(jnp.where(vals == m, iota, _NUM_EXPERTS),
                      axis=1, keepdims=True)
        top_vals.append(m)
        top_idx.append(idx)
        vals = jnp.where(iota == idx, -jnp.inf, vals)
    tv = jnp.concatenate(top_vals, axis=1)
    ti = jnp.concatenate(top_idx, axis=1)
    e = jnp.exp(tv - tv[:, 0:1])
    w = e / jnp.sum(e, axis=1, keepdims=True)
    idx_ref[...] = ti
    w_ref[...] = w


@jax.jit
def kernel(x, W_in, b_in, tau, A, B, W_gate, b_gate):
    del tau, A
    n_tokens, hidden = x.shape
    wt = W_in.T
    wg = W_gate.T
    b_in2 = b_in.reshape(1, _ROUTER_DIM)
    b_gate2 = b_gate.reshape(1, _NUM_EXPERTS)
    grid = (n_tokens // _BLK,)
    out_idx, out_w = pl.pallas_call(
        _router_kernel,
        grid=grid,
        in_specs=[
            pl.BlockSpec((_BLK, hidden), lambda i: (i, 0)),
            pl.BlockSpec((hidden, _ROUTER_DIM), lambda i: (0, 0)),
            pl.BlockSpec((1, _ROUTER_DIM), lambda i: (0, 0)),
            pl.BlockSpec((_ROUTER_DIM, _ROUTER_DIM), lambda i: (0, 0)),
            pl.BlockSpec((_ROUTER_DIM, _NUM_EXPERTS), lambda i: (0, 0)),
            pl.BlockSpec((1, _NUM_EXPERTS), lambda i: (0, 0)),
        ],
        out_specs=[
            pl.BlockSpec((_BLK, _TOP_K), lambda i: (i, 0)),
            pl.BlockSpec((_BLK, _TOP_K), lambda i: (i, 0)),
        ],
        out_shape=[
            jax.ShapeDtypeStruct((n_tokens, _TOP_K), jnp.int32),
            jax.ShapeDtypeStruct((n_tokens, _TOP_K), jnp.float32),
        ],
        compiler_params=pltpu.CompilerParams(
            dimension_semantics=("parallel",),
        ),
    )(x, wt, b_in2, B, wg, b_gate2)
    return out_idx, out_w


# speedup vs baseline: 1.2820x; 1.1370x over previous
import functools

import jax
import jax.numpy as jnp
from jax.experimental import pallas as pl
from jax.experimental.pallas import tpu as pltpu

_ROUTER_DIM = 256
_NUM_EXPERTS = 64
_TOP_K = 8
_BLK = 1024


def _router_kernel(x_ref, wt_ref, b_in_ref, b_mat_ref, wg_ref, b_gate_ref,
                   idx_ref, w_ref):
    xb = x_ref[...]
    xp = jnp.dot(xb, wt_ref[...], preferred_element_type=jnp.float32)
    xp = xp + b_in_ref[...]
    h = jnp.tanh(jnp.dot(xp, b_mat_ref[...],
                         preferred_element_type=jnp.float32)) * 0.1
    logits = jnp.dot(h, wg_ref[...], preferred_element_type=jnp.float32)
    logits = logits + b_gate_ref[...]

    blk = logits.shape[0]
    y = jax.lax.bitcast_convert_type(logits, jnp.int32)
    s = y ^ ((y >> 31) & jnp.int32(0x7FFFFFFF))
    iota = jax.lax.broadcasted_iota(jnp.int32, (blk, _NUM_EXPERTS), 1)
    key = (s & jnp.int32(~63)) | (63 - iota)
    tops = []
    for _ in range(_TOP_K):
        m = jnp.max(key, axis=1, keepdims=True)
        tops.append(m)
        key = jnp.where(key == m, jnp.int32(-(2**31)), key)
    tk = jnp.concatenate(tops, axis=1)
    ti = 63 - (tk & 63)
    y2 = tk & jnp.int32(~63)
    y2 = y2 ^ ((y2 >> 31) & jnp.int32(0x7FFFFFFF))
    tv = jax.lax.bitcast_convert_type(y2, jnp.float32)
    e = jnp.exp(tv - tv[:, 0:1])
    w = e / jnp.sum(e, axis=1, keepdims=True)
    idx_ref[...] = ti
    w_ref[...] = w


@jax.jit
def kernel(x, W_in, b_in, tau, A, B, W_gate, b_gate):
    del tau, A
    n_tokens, hidden = x.shape
    wt = W_in.T
    wg = W_gate.T
    b_in2 = b_in.reshape(1, _ROUTER_DIM)
    b_gate2 = b_gate.reshape(1, _NUM_EXPERTS)
    grid = (n_tokens // _BLK,)
    out_idx, out_w = pl.pallas_call(
        _router_kernel,
        grid=grid,
        in_specs=[
            pl.BlockSpec((_BLK, hidden), lambda i: (i, 0)),
            pl.BlockSpec((hidden, _ROUTER_DIM), lambda i: (0, 0)),
            pl.BlockSpec((1, _ROUTER_DIM), lambda i: (0, 0)),
            pl.BlockSpec((_ROUTER_DIM, _ROUTER_DIM), lambda i: (0, 0)),
            pl.BlockSpec((_ROUTER_DIM, _NUM_EXPERTS), lambda i: (0, 0)),
            pl.BlockSpec((1, _NUM_EXPERTS), lambda i: (0, 0)),
        ],
        out_specs=[
            pl.BlockSpec((_BLK, _TOP_K), lambda i: (i, 0)),
            pl.BlockSpec((_BLK, _TOP_K), lambda i: (i, 0)),
        ],
        out_shape=[
            jax.ShapeDtypeStruct((n_tokens, _TOP_K), jnp.int32),
            jax.ShapeDtypeStruct((n_tokens, _TOP_K), jnp.float32),
        ],
        compiler_params=pltpu.CompilerParams(
            dimension_semantics=("parallel",),
        ),
    )(x, wt, b_in2, B, wg, b_gate2)
    return out_idx, out_w


# speedup vs baseline: 1.3934x; 1.0868x over previous
import dataclasses
import functools

import jax
import jax.numpy as jnp
from jax import lax
from jax.experimental import pallas as pl
from jax.experimental.pallas import tpu as pltpu
from jax.experimental.pallas import tpu_sc as plsc

_ROUTER_DIM = 256
_NUM_EXPERTS = 64
_TOP_K = 8
_BLK = 1024

_SC_CORES = 2
_SC_SUBCORES = 16
_SC_WORKERS = _SC_CORES * _SC_SUBCORES
_LANES = 16


def _logits_kernel(x_ref, wt_ref, b_in_ref, b_mat_ref, wg_ref, b_gate_ref,
                   out_ref):
    xb = x_ref[...]
    xp = jnp.dot(xb, wt_ref[...], preferred_element_type=jnp.float32)
    xp = xp + b_in_ref[...]
    h = jnp.tanh(jnp.dot(xp, b_mat_ref[...],
                         preferred_element_type=jnp.float32)) * 0.1
    logits = jnp.dot(h, wg_ref[...], preferred_element_type=jnp.float32)
    out_ref[...] = logits + b_gate_ref[...]


def _tc_logits(x, wt, b_in2, B, wg, b_gate2):
    n_tokens, hidden = x.shape
    grid = (n_tokens // _BLK,)
    return pl.pallas_call(
        _logits_kernel,
        grid=grid,
        in_specs=[
            pl.BlockSpec((_BLK, hidden), lambda i: (i, 0)),
            pl.BlockSpec((hidden, _ROUTER_DIM), lambda i: (0, 0)),
            pl.BlockSpec((1, _ROUTER_DIM), lambda i: (0, 0)),
            pl.BlockSpec((_ROUTER_DIM, _ROUTER_DIM), lambda i: (0, 0)),
            pl.BlockSpec((_ROUTER_DIM, _NUM_EXPERTS), lambda i: (0, 0)),
            pl.BlockSpec((1, _NUM_EXPERTS), lambda i: (0, 0)),
        ],
        out_specs=pl.BlockSpec((_BLK, _NUM_EXPERTS), lambda i: (i, 0)),
        out_shape=jax.ShapeDtypeStruct((n_tokens, _NUM_EXPERTS), jnp.float32),
        compiler_params=pltpu.CompilerParams(
            dimension_semantics=("parallel",),
        ),
    )(x, wt, b_in2, B, wg, b_gate2)


def _sc_topk(logits):
    n = logits.shape[0]
    rows_per_worker = n // _SC_WORKERS
    rows_per_chunk = 256
    n_chunks = rows_per_worker // rows_per_chunk
    mesh = plsc.VectorSubcoreMesh(core_axis_name="c", subcore_axis_name="s",
                                  num_cores=_SC_CORES,
                                  num_subcores=_SC_SUBCORES)
    cp = pltpu.CompilerParams()
    if "needs_layout_passes" in pltpu.CompilerParams.__dataclass_fields__:
        cp = dataclasses.replace(cp, needs_layout_passes=False)

    @functools.partial(
        pl.kernel, mesh=mesh,
        out_type=(jax.ShapeDtypeStruct((n, _LANES), jnp.int32),
                  jax.ShapeDtypeStruct((n, _LANES), jnp.float32)),
        scratch_types=[
            pltpu.VMEM((rows_per_chunk, _NUM_EXPERTS), jnp.float32),
            pltpu.VMEM((rows_per_chunk, _LANES), jnp.int32),
            pltpu.VMEM((rows_per_chunk, _LANES), jnp.float32),
        ],
        compiler_params=cp,
    )
    def k(l_hbm, idx_hbm, w_hbm, l_v, i_v, w_v):
        wid = lax.axis_index("s") * _SC_CORES + lax.axis_index("c")
        iota16 = lax.iota(jnp.int32, _LANES)
        lane_lt8 = iota16 < 8

        @pl.loop(0, n_chunks)
        def _(ci):
            base = wid * rows_per_worker + ci * rows_per_chunk
            pltpu.sync_copy(l_hbm.at[pl.ds(base, rows_per_chunk)], l_v)

            @pl.loop(0, rows_per_chunk)
            def _(r):
                merged = []
                for c in range(4):
                    v = l_v[r, pl.ds(c * _LANES, _LANES)]
                    y = lax.bitcast_convert_type(v, jnp.int32)
                    s = y ^ ((y >> 31) & jnp.int32(0x7FFFFFFF))
                    key = (s & jnp.int32(~63)) | (63 - (iota16 + c * _LANES))
                    sk, _unused = plsc.sort_key_val(key, key, descending=True)
                    merged.append(sk)

                def merge(a, b):
                    comb = jnp.where(lane_lt8, a, lax.rev(b, (0,)))
                    sk, _unused = plsc.sort_key_val(comb, comb,
                                                    descending=True)
                    return sk

                fin = merge(merge(merged[0], merged[1]),
                            merge(merged[2], merged[3]))
                idx = 63 - (fin & 63)
                y2 = fin & jnp.int32(~63)
                y2 = y2 ^ ((y2 >> 31) & jnp.int32(0x7FFFFFFF))
                tv = lax.bitcast_convert_type(y2, jnp.float32)
                m = jnp.max(tv)
                e = jnp.where(lane_lt8, jnp.exp(tv - m), 0.0)
                w = e / jnp.sum(e)
                i_v[r, :] = idx
                w_v[r, :] = w

            pltpu.sync_copy(i_v, idx_hbm.at[pl.ds(base, rows_per_chunk)])
            pltpu.sync_copy(w_v, w_hbm.at[pl.ds(base, rows_per_chunk)])

    return k(logits)


@jax.jit
def kernel(x, W_in, b_in, tau, A, B, W_gate, b_gate):
    del tau, A
    wt = W_in.T
    wg = W_gate.T
    b_in2 = b_in.reshape(1, _ROUTER_DIM)
    b_gate2 = b_gate.reshape(1, _NUM_EXPERTS)
    logits = _tc_logits(x, wt, b_in2, B, wg, b_gate2)
    idx16, w16 = _sc_topk(logits)
    return idx16[:, :_TOP_K], w16[:, :_TOP_K]
